# baseline (speedup 1.0000x reference)
import jax
import jax.numpy as jnp
from jax.experimental import pallas as pl
from jax.experimental.pallas import tpu as pltpu

N = 20000
NPAD = 20480
R = NPAD // 128
K = 2000
KPAD = 2048
NBLK = KPAD // 128
MAX_DET = 500
NMS_THR = 0.7
SCORE_THR = 0.1
NEG = -1e9
WIN = 136
CPAD = KPAD + WIN + 8


def _select_kernel(xs_ref, ys_ref, ws_ref, hs_ref, s_ref, comp_ref,
                   x1s, y1s, x2s, y2s, svs, poss, sels):
    f32 = jnp.float32
    i32 = jnp.int32
    bf16 = jnp.bfloat16
    hi = jax.lax.Precision.HIGHEST

    s = s_ref[...]
    sv = jnp.where(s > SCORE_THR, s, NEG)
    key = jax.lax.bitcast_convert_type(sv, i32)

    cx = xs_ref[...] * 1600.0
    cy = ys_ref[...] * 900.0
    w = ws_ref[...] * 120.0 + 4.0
    h = hs_ref[...] * 120.0 + 4.0
    x1s[...] = cx - w / 2.0
    y1s[...] = cy - h / 2.0
    x2s[...] = cx + w / 2.0
    y2s[...] = cy + h / 2.0
    svs[...] = sv

    def bs_body(_, carry):
        lo, bhi = carry
        mid = (lo + bhi) >> 1
        cnt = jnp.sum((key > mid).astype(f32))
        p = cnt >= float(K)
        return jnp.where(p, mid, lo), jnp.where(p, bhi, mid)

    lo0 = jnp.int32(-2147483647)
    hi0 = jnp.int32(1073741824)
    _, t_key = jax.lax.fori_loop(0, 32, bs_body, (lo0, hi0))
    cnt_gt = jnp.sum((key > t_key).astype(f32))
    need = float(K) - cnt_gt

    suba = jax.lax.broadcasted_iota(i32, (128, 128), 0)
    lanb = jax.lax.broadcasted_iota(i32, (128, 128), 1)
    lmat = (suba < lanb).astype(bf16)
    sub160 = jax.lax.broadcasted_iota(i32, (R, R), 0)
    lan160 = jax.lax.broadcasted_iota(i32, (R, R), 1)
    l160 = (lan160 < sub160).astype(bf16)
    dn = (((1,), (0,)), ((), ()))

    def prefix(mask_f32):
        lanepre = jax.lax.dot_general(mask_f32.astype(bf16), lmat, dn,
                                      preferred_element_type=f32)
        rt = jnp.sum(mask_f32, axis=1, keepdims=True)
        rowpre = jax.lax.dot_general(l160, rt.astype(bf16), dn,
                                     preferred_element_type=f32)
        return lanepre + rowpre

    eqm = (key == t_key).astype(f32)
    tie = prefix(eqm)
    sel = ((key > t_key) | ((key == t_key) & (tie < need))).astype(f32)
    poss[...] = prefix(sel)
    sels[...] = sel

    comp_ref[...] = jnp.zeros((CPAD, 8), f32)
    eye = (suba == lanb).astype(f32)
    iwin = jax.lax.broadcasted_iota(i32, (128, WIN), 1).astype(f32)
    subc = jax.lax.broadcasted_iota(i32, (128, 1), 0)
    dn1 = (((1,), (1,)), ((), ()))
    dn0 = (((0,), (0,)), ((), ()))

    def grp_body(g, carry):
        row0 = pl.multiple_of(g * 8, 8)
        stacked = jnp.concatenate(
            [r[pl.ds(row0, 8), :] for r in (x1s, y1s, x2s, y2s, svs, poss, sels)],
            axis=0)
        t = jax.lax.dot_general(eye, stacked, dn1, precision=hi,
                                preferred_element_type=f32)
        for u in range(8):
            pos_col = t[:, 40 + u:41 + u]
            sel_col = t[:, 48 + u:49 + u]
            base = jnp.sum(t[0:1, 40 + u:41 + u])
            bi = pl.multiple_of((base.astype(i32) >> 3) << 3, 8)
            posrel = pos_col - bi.astype(f32)
            oh = ((posrel == iwin) & (sel_col > 0.5)).astype(f32)
            gcol = ((g * 8 + u) * 128 + subc).astype(f32)
            vals = jnp.concatenate(
                [t[:, u:u + 1], t[:, 8 + u:9 + u], t[:, 16 + u:17 + u],
                 t[:, 24 + u:25 + u], t[:, 32 + u:33 + u], gcol,
                 jnp.zeros((128, 2), f32)], axis=1)
            contrib = jax.lax.dot_general(oh, vals, dn0, precision=hi,
                                          preferred_element_type=f32)
            comp_ref[pl.ds(bi, WIN), :] = comp_ref[pl.ds(bi, WIN), :] + contrib
        return carry

    jax.lax.fori_loop(0, R // 8, grp_body, 0)


def _nms_kernel(bb_ref, bbT_ref, ts_row_ref, ts_col_ref, g_row_ref,
                g_col_ref, out_ref, m_ref):
    f32 = jnp.float32
    bf16 = jnp.bfloat16
    sub = jax.lax.broadcasted_iota(jnp.int32, (128, 128), 0)
    lane = jax.lax.broadcasted_iota(jnp.int32, (128, 128), 1)

    for rb in range(NBLK):
        rows = slice(rb * 128, (rb + 1) * 128)
        x1r = bb_ref[rows, 0:1]
        y1r = bb_ref[rows, 1:2]
        x2r = bb_ref[rows, 2:3]
        y2r = bb_ref[rows, 3:4]
        area_r = (x2r - x1r) * (y2r - y1r)
        s_r = ts_col_ref[rows, 0:1]
        g_r = g_col_ref[rows, 0:1]
        for cb in range(NBLK):
            cols = slice(cb * 128, (cb + 1) * 128)
            x1c = bbT_ref[0:1, cols]
            y1c = bbT_ref[1:2, cols]
            x2c = bbT_ref[2:3, cols]
            y2c = bbT_ref[3:4, cols]
            wx = jnp.maximum(jnp.minimum(x2r, x2c) - jnp.maximum(x1r, x1c), 0.0)
            wy = jnp.maximum(jnp.minimum(y2r, y2c) - jnp.maximum(y1r, y1c), 0.0)
            inter = wx * wy
            area_c = (x2c - x1c) * (y2c - y1c)
            union = area_r + area_c - inter
            iou = inter / jnp.maximum(union, 1e-6)
            s_c = ts_row_ref[cb:cb + 1, :]
            g_c = g_row_ref[cb:cb + 1, :]
            beats = (s_r > s_c) | ((s_r == s_c) & (g_r < g_c))
            m_ref[rows, cols] = ((iou > NMS_THR) & beats).astype(bf16)

    def cond(carry):
        _, changed = carry
        return changed > 0

    def body(carry):
        k, _ = carry
        parts = []
        for jb in range(NBLK):
            mm = m_ref[:, jb * 128:(jb + 1) * 128]
            supp = jnp.dot(k, mm, preferred_element_type=f32)
            parts.append((supp == 0.0).astype(bf16))
        k_new = jnp.concatenate(parts, axis=1)
        changed = jnp.sum(jnp.abs((k_new - k).astype(f32)))
        return k_new, changed

    k0 = jnp.ones((1, KPAD), bf16)
    k_fin, _ = jax.lax.while_loop(cond, body, (k0, jnp.float32(1.0)))

    hi = jax.lax.Precision.HIGHEST
    f_rows = []
    for jc in range(NBLK):
        krow = k_fin[:, jc * 128:(jc + 1) * 128]
        tsr = ts_row_ref[jc:jc + 1, :]
        f_rows.append(jnp.where((krow > 0) & (tsr > NEG / 2), tsr, NEG))
    out_acc = jnp.zeros((4 * 128, 8), f32)
    for jc in range(NBLK):
        krow = k_fin[:, jc * 128:(jc + 1) * 128].astype(f32)
        kcol = jnp.sum(jnp.where(sub == lane, krow, 0.0), axis=1, keepdims=True)
        tsc = ts_col_ref[jc * 128:(jc + 1) * 128, 0:1]
        gc = g_col_ref[jc * 128:(jc + 1) * 128, 0:1]
        f_j = jnp.where((kcol > 0.5) & (tsc > NEG / 2), tsc, NEG)
        rank = jnp.zeros((128, 1), f32)
        for ic in range(NBLK):
            f_i = f_rows[ic]
            tsr_i = ts_row_ref[ic:ic + 1, :]
            gr_i = g_row_ref[ic:ic + 1, :]
            pbeats = (tsr_i > tsc) | ((tsr_i == tsc) & (gr_i < gc))
            beats = (f_i > f_j) | ((f_i == f_j) & pbeats)
            rank = rank + jnp.sum(beats.astype(f32), axis=1, keepdims=True)
        ioh = jax.lax.broadcasted_iota(jnp.int32, (128, 4 * 128), 1).astype(f32)
        oh = (rank == ioh).astype(f32)
        vals = jnp.concatenate(
            [bb_ref[jc * 128:(jc + 1) * 128, :], f_j, jnp.zeros((128, 3), f32)],
            axis=1)
        out_acc = out_acc + jax.lax.dot_general(
            oh, vals, (((0,), (0,)), ((), ())), precision=hi,
            preferred_element_type=f32)
    out_ref[...] = out_acc


def kernel(boxes, scores):
    f32 = jnp.float32
    boxes_p = jnp.pad(boxes, ((0, NPAD - N), (0, 0)))
    scores_p = jnp.pad(scores, (0, NPAD - N), constant_values=0.0)
    xs = boxes_p[:, 0].reshape(R, 128)
    ys = boxes_p[:, 1].reshape(R, 128)
    ws = boxes_p[:, 2].reshape(R, 128)
    hs = boxes_p[:, 3].reshape(R, 128)
    sp = scores_p.reshape(R, 128)
    comp = pl.pallas_call(
        _select_kernel,
        out_shape=jax.ShapeDtypeStruct((CPAD, 8), f32),
        scratch_shapes=[pltpu.VMEM((R, 128), f32)] * 7,
    )(xs, ys, ws, hs, sp)
    pad = KPAD - K
    bb_p = jnp.concatenate([comp[:K, :4], jnp.zeros((pad, 4), f32)], axis=0)
    ts = jnp.concatenate([comp[:K, 4], jnp.full((pad,), NEG, f32)])
    gg = jnp.concatenate([comp[:K, 5], jnp.full((pad,), 1e9, f32)])
    out = pl.pallas_call(
        _nms_kernel,
        out_shape=jax.ShapeDtypeStruct((4 * 128, 8), f32),
        scratch_shapes=[pltpu.VMEM((KPAD, KPAD), jnp.bfloat16)],
    )(bb_p, bb_p.T, ts.reshape(NBLK, 128), ts.reshape(KPAD, 1),
      gg.reshape(NBLK, 128), gg.reshape(KPAD, 1))
    return out[:MAX_DET, :5]
